# baseline (speedup 1.0000x reference)
import functools
import math

import jax
import jax.numpy as jnp
from jax import lax
from jax.experimental import pallas as pl
from jax.experimental.pallas import tpu as pltpu
from jax.experimental.pallas import tpu_sc as plsc

D_MODEL = 1024
SCALE = math.sqrt(D_MODEL)

NC = 2
NS = 16
NW = NC * NS

LANES = 16
CHUNK = 32


def _emb_body(x_hbm, table_hbm, out_hbm, idx_v, rows_v, gsem):
    n_chunks = x_hbm.shape[1]
    wid = lax.axis_index("s") * NC + lax.axis_index("c")
    b_per_w = n_chunks * CHUNK
    base_row = wid * b_per_w

    pltpu.sync_copy(x_hbm.at[wid], idx_v)

    def chunk_body(g, carry):
        pltpu.async_copy(table_hbm.at[idx_v.at[g]], rows_v, gsem).wait()

        def row_body(r, c):
            for j in range(D_MODEL // LANES):
                sl = pl.ds(j * LANES, LANES)
                rows_v[r, sl] = rows_v[r, sl] * SCALE
            return c

        lax.fori_loop(0, CHUNK, row_body, 0)

        pltpu.sync_copy(rows_v, out_hbm.at[pl.ds(base_row + g * CHUNK, CHUNK)])
        return carry

    lax.fori_loop(0, n_chunks, chunk_body, 0)


def _build(batch_seq):
    n_chunks = batch_seq // (NW * CHUNK)
    mesh = plsc.VectorSubcoreMesh(core_axis_name="c", subcore_axis_name="s")
    return functools.partial(
        pl.kernel,
        out_type=jax.ShapeDtypeStruct((batch_seq, D_MODEL), jnp.float32),
        mesh=mesh,
        scratch_types=[
            pltpu.VMEM((n_chunks, CHUNK), jnp.int32),
            pltpu.VMEM((CHUNK, D_MODEL), jnp.float32),
            pltpu.SemaphoreType.DMA,
        ],
    )(_emb_body)


@jax.jit
def kernel(x, table):
    b, s = x.shape
    batch_seq = b * s
    xw = x.reshape(NW, batch_seq // (NW * CHUNK), CHUNK).astype(jnp.int32)
    out = _build(batch_seq)(xw, table)
    return out.reshape(b, s, D_MODEL)


# speedup vs baseline: 1.6703x; 1.6703x over previous
import functools
import math

import jax
import jax.numpy as jnp
from jax import lax
from jax.experimental import pallas as pl
from jax.experimental.pallas import tpu as pltpu
from jax.experimental.pallas import tpu_sc as plsc

D_MODEL = 1024
SCALE = math.sqrt(D_MODEL)

NC = 2
NS = 16
NW = NC * NS

LANES = 16
CHUNK = 16
NBUF = 2


def _emb_body(x_hbm, table_hbm, out_hbm, idx_v, ib0, ib1, ob0, ob1,
              gs0, gs1, os0, os1):
    n_chunks = x_hbm.shape[1]
    ibufs, obufs = (ib0, ib1), (ob0, ob1)
    gsems, osems = (gs0, gs1), (os0, os1)

    wid = lax.axis_index("s") * NC + lax.axis_index("c")
    b_per_w = n_chunks * CHUNK
    base_row = wid * b_per_w

    pltpu.sync_copy(x_hbm.at[wid], idx_v)

    for b in range(NBUF):
        pltpu.async_copy(table_hbm.at[idx_v.at[b]], ibufs[b], gsems[b])

    def scale_chunk(ib, ob):
        def row_body(r, c):
            for j in range(D_MODEL // LANES):
                sl = pl.ds(j * LANES, LANES)
                ob[r, sl] = ib[r, sl] * SCALE
            return c
        lax.fori_loop(0, CHUNK, row_body, 0)

    def pair_body(i, carry):
        g0 = i * NBUF
        for b in range(NBUF):
            g = g0 + b
            pltpu.make_async_copy(
                table_hbm.at[pl.ds(0, CHUNK)], ibufs[b], gsems[b]).wait()

            @pl.when(g >= NBUF)
            def _():
                pltpu.make_async_copy(
                    obufs[b], out_hbm.at[pl.ds(0, CHUNK)], osems[b]).wait()

            scale_chunk(ibufs[b], obufs[b])

            pltpu.async_copy(
                obufs[b], out_hbm.at[pl.ds(base_row + g * CHUNK, CHUNK)],
                osems[b])

            @pl.when(g + NBUF < n_chunks)
            def _():
                pltpu.async_copy(
                    table_hbm.at[idx_v.at[g + NBUF]], ibufs[b], gsems[b])
        return carry

    lax.fori_loop(0, n_chunks // NBUF, pair_body, 0)

    for b in range(NBUF):
        pltpu.make_async_copy(
            obufs[b], out_hbm.at[pl.ds(0, CHUNK)], osems[b]).wait()


def _build(batch_seq):
    n_chunks = batch_seq // (NW * CHUNK)
    mesh = plsc.VectorSubcoreMesh(core_axis_name="c", subcore_axis_name="s")
    buf = pltpu.VMEM((CHUNK, D_MODEL), jnp.float32)
    return functools.partial(
        pl.kernel,
        out_type=jax.ShapeDtypeStruct((batch_seq, D_MODEL), jnp.float32),
        mesh=mesh,
        scratch_types=[
            pltpu.VMEM((n_chunks, CHUNK), jnp.int32),
            buf, buf, buf, buf,
            pltpu.SemaphoreType.DMA, pltpu.SemaphoreType.DMA,
            pltpu.SemaphoreType.DMA, pltpu.SemaphoreType.DMA,
        ],
    )(_emb_body)


@jax.jit
def kernel(x, table):
    b, s = x.shape
    batch_seq = b * s
    xw = x.reshape(NW, batch_seq // (NW * CHUNK), CHUNK).astype(jnp.int32)
    out = _build(batch_seq)(xw, table)
    return out.reshape(b, s, D_MODEL)


# speedup vs baseline: 1.7468x; 1.0458x over previous
import functools
import math

import jax
import jax.numpy as jnp
from jax import lax
from jax.experimental import pallas as pl
from jax.experimental.pallas import tpu as pltpu
from jax.experimental.pallas import tpu_sc as plsc

D_MODEL = 1024
SCALE = math.sqrt(D_MODEL)

NC = 2
NS = 16
NW = NC * NS

LANES = 16
CHUNK = 32
NBUF = 3


def _emb_body(x_hbm, table_hbm, out_hbm, idx_v, b0, b1, b2,
              gs0, gs1, gs2, os0, os1, os2):
    n_chunks = x_hbm.shape[1]
    bufs = (b0, b1, b2)
    gsems = (gs0, gs1, gs2)
    osems = (os0, os1, os2)

    wid = lax.axis_index("s") * NC + lax.axis_index("c")
    b_per_w = n_chunks * CHUNK
    base_row = wid * b_per_w

    pltpu.sync_copy(x_hbm.at[wid], idx_v)

    pltpu.async_copy(table_hbm.at[idx_v.at[0]], bufs[0], gsems[0])
    pltpu.async_copy(table_hbm.at[idx_v.at[1]], bufs[1], gsems[1])

    def wait_gather(b):
        pltpu.make_async_copy(
            table_hbm.at[pl.ds(0, CHUNK)], bufs[b], gsems[b]).wait()

    def wait_write(b):
        pltpu.make_async_copy(
            bufs[b], out_hbm.at[pl.ds(0, CHUNK)], osems[b]).wait()

    def scale_chunk(buf):
        def row_body(r, c):
            for j in range(D_MODEL // LANES):
                sl = pl.ds(j * LANES, LANES)
                buf[r, sl] = buf[r, sl] * SCALE
            return c
        lax.fori_loop(0, CHUNK, row_body, 0)

    def slot(g, b, refill_mode):
        wait_gather(b)
        scale_chunk(bufs[b])
        pltpu.async_copy(
            bufs[b], out_hbm.at[pl.ds(base_row + g * CHUNK, CHUNK)], osems[b])
        nb = (b + 2) % NBUF

        def refill(wait=True):
            if wait:
                wait_write(nb)
            pltpu.async_copy(table_hbm.at[idx_v.at[g + 2]], bufs[nb], gsems[nb])

        if refill_mode == "first":
            refill(wait=False)
        elif refill_mode == "dyn":
            pl.when(g + 2 < n_chunks)(refill)
        elif refill_mode == "static":
            if g + 2 < n_chunks:
                refill()

    slot(0, 0, "first")
    n_ring = (n_chunks - 1 - 2) // NBUF

    def ring_body(i, carry):
        g0 = 1 + i * NBUF
        for k in range(NBUF):
            slot(g0 + k, (1 + k) % NBUF, "dyn")
        return carry

    lax.fori_loop(0, n_ring, ring_body, 0)
    for g in range(1 + n_ring * NBUF, n_chunks):
        slot(g, g % NBUF, "static")

    for b in range(NBUF):
        wait_write(b)


def _build(batch_seq):
    n_chunks = batch_seq // (NW * CHUNK)
    mesh = plsc.VectorSubcoreMesh(core_axis_name="c", subcore_axis_name="s")
    buf = pltpu.VMEM((CHUNK, D_MODEL), jnp.float32)
    sem = pltpu.SemaphoreType.DMA
    return functools.partial(
        pl.kernel,
        out_type=jax.ShapeDtypeStruct((batch_seq, D_MODEL), jnp.float32),
        mesh=mesh,
        scratch_types=[
            pltpu.VMEM((n_chunks, CHUNK), jnp.int32),
            buf, buf, buf,
            sem, sem, sem, sem, sem, sem,
        ],
    )(_emb_body)


@jax.jit
def kernel(x, table):
    b, s = x.shape
    batch_seq = b * s
    xw = x.reshape(NW, batch_seq // (NW * CHUNK), CHUNK).astype(jnp.int32)
    out = _build(batch_seq)(xw, table)
    return out.reshape(b, s, D_MODEL)
